# Initial kernel scaffold; baseline (speedup 1.0000x reference)
import jax
import jax.numpy as jnp
from jax.experimental import pallas as pl


def kernel(image, fx, fy, autoenc_feats, labels, bn_coords_g, bn_coords_b, c1_w, c1_b, c2_w, c2_b, bn_reduc_g, bn_reduc_b, sr_w, sr_b, bn512_g, bn512_b, m_w, m_b):
    raise NotImplementedError("write your pallas kernel here")


# speedup vs baseline: 1.4580x; 1.4580x over previous
import functools

import jax
import jax.numpy as jnp
from jax import lax
from jax.experimental import pallas as pl
from jax.experimental.pallas import tpu as pltpu
from jax.experimental.pallas import tpu_sc as plsc

_B = 2
_C = 256
_H = 384
_W = 384
_HW = _H * _W
_S = 1200
_NSEG = _B * _S
_PCH = 4096
_NCHUNK = _HW // _PCH
_NG = 16
_GC = 16


@functools.partial(
    pl.kernel,
    out_type=[
        jax.ShapeDtypeStruct((_B * _NG, 1, _S * _GC), jnp.float32),
        jax.ShapeDtypeStruct((_B * _NG, 1, 5 * _S), jnp.float32),
    ],
    scratch_types=[
        pltpu.VMEM((_GC, _PCH), jnp.float32),
        pltpu.VMEM((_PCH,), jnp.int32),
        pltpu.VMEM((_PCH,), jnp.float32),
        pltpu.VMEM((_PCH,), jnp.float32),
        pltpu.VMEM((_PCH,), jnp.float32),
        pltpu.VMEM((_PCH,), jnp.float32),
        pltpu.VMEM((_S * _GC,), jnp.float32),
        pltpu.VMEM((5 * _S,), jnp.float32),
    ],
    mesh=plsc.VectorSubcoreMesh(core_axis_name="c", subcore_axis_name="s"),
    compiler_params=pltpu.CompilerParams(needs_layout_passes=False),
)
def _sc_pool(feats_hbm, labs_hbm, xx_hbm, yy_hbm, fx_hbm, fy_hbm,
             feat_out, aux_out,
             fbuf, lbuf, xbuf, ybuf, fxbuf, fybuf, acc, aux):
    bb = lax.axis_index("c")
    g = lax.axis_index("s")
    wid = bb * _NG + g
    p0 = bb * _C + g * _GC

    zeros16 = jnp.zeros((16,), jnp.float32)
    ones16 = jnp.ones((16,), jnp.float32)

    def _zacc(i, carry):
        acc[pl.ds(i * 16, 16)] = zeros16
        return carry
    lax.fori_loop(0, (_S * _GC) // 16, _zacc, 0)

    def _zaux(i, carry):
        aux[pl.ds(i * 16, 16)] = zeros16
        return carry
    lax.fori_loop(0, (5 * _S) // 16, _zaux, 0)

    def chunk_body(ci, carry):
        s = ci * _PCH
        pltpu.sync_copy(feats_hbm.at[pl.ds(p0, _GC), pl.ds(s, _PCH)], fbuf)
        pltpu.sync_copy(labs_hbm.at[bb, pl.ds(s, _PCH)], lbuf)

        def blk(pb, inner):
            base = pb * 16
            lab16 = lbuf[pl.ds(base, 16)] * _GC
            for c in range(_GC):
                v = fbuf[c, pl.ds(base, 16)]
                plsc.addupdate_scatter(acc, [lab16 + c], v)
            return inner
        lax.fori_loop(0, _PCH // 16, blk, 0)

        @pl.when(lax.rem(ci, _NG) == g)
        def _aux_work():
            pltpu.sync_copy(xx_hbm.at[pl.ds(s, _PCH)], xbuf)
            pltpu.sync_copy(yy_hbm.at[pl.ds(s, _PCH)], ybuf)
            pltpu.sync_copy(fx_hbm.at[bb, pl.ds(s, _PCH)], fxbuf)
            pltpu.sync_copy(fy_hbm.at[bb, pl.ds(s, _PCH)], fybuf)

            def ablk(pb, inner):
                base = pb * 16
                lab = lbuf[pl.ds(base, 16)]
                plsc.addupdate_scatter(aux, [lab], xbuf[pl.ds(base, 16)])
                plsc.addupdate_scatter(aux, [lab + _S], ybuf[pl.ds(base, 16)])
                plsc.addupdate_scatter(aux, [lab + 2 * _S], fxbuf[pl.ds(base, 16)])
                plsc.addupdate_scatter(aux, [lab + 3 * _S], fybuf[pl.ds(base, 16)])
                plsc.addupdate_scatter(aux, [lab + 4 * _S], ones16)
                return inner
            lax.fori_loop(0, _PCH // 16, ablk, 0)
        return carry

    lax.fori_loop(0, _NCHUNK, chunk_body, 0)

    pltpu.sync_copy(acc, feat_out.at[wid, 0])
    pltpu.sync_copy(aux, aux_out.at[wid, 0])


def _dense_body(feat_ref, aux_ref, cg_ref, cb_ref, c1wt_ref, c1b_ref,
                c2wt_ref, c2b_ref, brg_ref, brb_ref, srwt_ref, srb_ref,
                b5g_ref, b5b_ref, mwt_ref, mb_ref, siam_ref, pos_ref):
    aux = jnp.sum(aux_ref[...], axis=0)

    def seg(k):
        return aux[k]

    xx_s, yy_s, fx_s, fy_s, cnt = seg(0), seg(1), seg(2), seg(3), seg(4)
    cts = jnp.maximum(cnt, 1.0)
    xx = xx_s / cts
    yy = yy_s / cts
    fxp = fx_s / cts
    fyp = fy_s / cts
    pooled = feat_ref[...] / cts[:, None]

    cg = cg_ref[...]
    cb = cb_ref[...]

    def bn1(v, gamma, beta):
        mu = jnp.mean(v)
        var = jnp.mean((v - mu) ** 2)
        return (v - mu) / jnp.sqrt(var + 1e-5) * gamma + beta

    c0 = bn1(xx, cg[0], cb[0])
    c1 = bn1(yy, cg[1], cb[1])
    c2 = bn1(fxp, cg[2], cb[2])
    c3 = bn1(fyp, cg[3], cb[3])

    w1 = c1wt_ref[...]
    x1 = (c0[:, None] * w1[0][None, :] + c1[:, None] * w1[1][None, :]
          + c2[:, None] * w1[2][None, :] + c3[:, None] * w1[3][None, :]
          + c1b_ref[...][None, :])
    x2 = jnp.dot(x1, c2wt_ref[...], preferred_element_type=jnp.float32)
    x2 = jnp.maximum(x2 + c2b_ref[...][None, :], 0.0)

    mu_c = jnp.mean(pooled, axis=0)
    var_c = jnp.mean((pooled - mu_c[None, :]) ** 2, axis=0)
    psn = ((pooled - mu_c[None, :]) / jnp.sqrt(var_c + 1e-5)
           * brg_ref[...][None, :] + brb_ref[...][None, :])

    r = jnp.dot(psn, srwt_ref[...], preferred_element_type=jnp.float32)
    r = r + srb_ref[...][None, :]
    mu_r = jnp.mean(r, axis=0)
    var_r = jnp.mean((r - mu_r[None, :]) ** 2, axis=0)
    rn = ((r - mu_r[None, :]) / jnp.sqrt(var_r + 1e-5)
          * b5g_ref[...][None, :] + b5b_ref[...][None, :])
    rn = jnp.maximum(rn, 0.0)

    mwt = mwt_ref[...]
    siam = (jnp.dot(x2, mwt[0:64], preferred_element_type=jnp.float32)
            + jnp.dot(rn, mwt[64:576], preferred_element_type=jnp.float32)
            + mb_ref[...][None, :])
    siam_ref[...] = siam
    pos_ref[...] = jnp.concatenate([xx[:, None], yy[:, None]], axis=1)


_dense_call = pl.pallas_call(
    _dense_body,
    out_shape=[
        jax.ShapeDtypeStruct((_NSEG, _C), jnp.float32),
        jax.ShapeDtypeStruct((_NSEG, 2), jnp.float32),
    ],
)


def kernel(image, fx, fy, autoenc_feats, labels, bn_coords_g, bn_coords_b,
           c1_w, c1_b, c2_w, c2_b, bn_reduc_g, bn_reduc_b, sr_w, sr_b,
           bn512_g, bn512_b, m_w, m_b):
    feats2 = autoenc_feats.reshape(_B * _C, _HW)
    labs = labels.reshape(_B, _HW)
    fxf = fx.reshape(_B, _HW)
    fyf = fy.reshape(_B, _HW)
    ar = jnp.arange(_H, dtype=jnp.float32) / (_H - 1)
    xxv = jnp.broadcast_to(ar[:, None], (_H, _W)).reshape(_HW)
    yyv = jnp.broadcast_to(ar[None, :], (_H, _W)).reshape(_HW)

    feat_sums, aux_raw = _sc_pool(feats2, labs, xxv, yyv, fxf, fyf)
    feat_sums = (feat_sums.reshape(_B, _NG, _S, _GC)
                 .transpose(0, 2, 1, 3).reshape(_NSEG, _C))
    aux_raw = (aux_raw.reshape(_B, _NG, 5, _S)
               .transpose(1, 2, 0, 3).reshape(_NG, 5, _NSEG))

    siam, pos = _dense_call(
        feat_sums, aux_raw,
        bn_coords_g, bn_coords_b, c1_w.T, c1_b, c2_w.T, c2_b,
        bn_reduc_g, bn_reduc_b, sr_w.T, sr_b, bn512_g, bn512_b,
        m_w.T, m_b)
    return siam, pos


# speedup vs baseline: 2.3408x; 1.6054x over previous
import functools

import jax
import jax.numpy as jnp
from jax import lax
from jax.experimental import pallas as pl
from jax.experimental.pallas import tpu as pltpu
from jax.experimental.pallas import tpu_sc as plsc

_B = 2
_C = 256
_H = 384
_W = 384
_HW = _H * _W
_S = 1200
_NSEG = _B * _S
_PCH = 4096
_NCHUNK = _HW // _PCH
_NG = 16
_GC = 16


@functools.partial(
    pl.kernel,
    out_type=[
        jax.ShapeDtypeStruct((_B * _NG, 1, _GC * _S), jnp.float32),
        jax.ShapeDtypeStruct((_B * _NG, 1, 5 * _S), jnp.float32),
    ],
    scratch_types=[
        pltpu.VMEM((_GC, _PCH), jnp.float32),
        pltpu.VMEM((_PCH,), jnp.int32),
        pltpu.VMEM((_PCH,), jnp.float32),
        pltpu.VMEM((_PCH,), jnp.float32),
        pltpu.VMEM((_PCH,), jnp.float32),
        pltpu.VMEM((_PCH,), jnp.float32),
        pltpu.VMEM((_GC * _S,), jnp.float32),
        pltpu.VMEM((5 * _S,), jnp.float32),
    ],
    mesh=plsc.VectorSubcoreMesh(core_axis_name="c", subcore_axis_name="s"),
    compiler_params=pltpu.CompilerParams(needs_layout_passes=False),
)
def _sc_pool(feats_hbm, labs_hbm, xx_hbm, yy_hbm, fx_hbm, fy_hbm,
             feat_out, aux_out,
             fbuf, lbuf, xbuf, ybuf, fxbuf, fybuf, acc, aux):
    bb = lax.axis_index("c")
    g = lax.axis_index("s")
    wid = bb * _NG + g
    p0 = bb * _C + g * _GC

    zeros16 = jnp.zeros((16,), jnp.float32)
    ones16 = jnp.ones((16,), jnp.float32)

    def _zacc(i, carry):
        acc[pl.ds(i * 16, 16)] = zeros16
        return carry
    lax.fori_loop(0, (_GC * _S) // 16, _zacc, 0, unroll=8)

    def _zaux(i, carry):
        aux[pl.ds(i * 16, 16)] = zeros16
        return carry
    lax.fori_loop(0, (5 * _S) // 16, _zaux, 0, unroll=8)

    def chunk_body(ci, carry):
        s = ci * _PCH
        pltpu.sync_copy(feats_hbm.at[pl.ds(p0, _GC), pl.ds(s, _PCH)], fbuf)
        pltpu.sync_copy(labs_hbm.at[bb, pl.ds(s, _PCH)], lbuf)

        def blk(pb, inner):
            base = pb * 16
            lab = lbuf[pl.ds(base, 16)]
            for c in range(_GC):
                v = fbuf[c, pl.ds(base, 16)]
                plsc.addupdate_scatter(acc, [lab + (c * _S)], v)
            return inner
        lax.fori_loop(0, _PCH // 16, blk, 0, unroll=4)

        @pl.when(lax.rem(ci, _NG) == g)
        def _aux_work():
            pltpu.sync_copy(xx_hbm.at[pl.ds(s, _PCH)], xbuf)
            pltpu.sync_copy(yy_hbm.at[pl.ds(s, _PCH)], ybuf)
            pltpu.sync_copy(fx_hbm.at[bb, pl.ds(s, _PCH)], fxbuf)
            pltpu.sync_copy(fy_hbm.at[bb, pl.ds(s, _PCH)], fybuf)

            def ablk(pb, inner):
                base = pb * 16
                lab = lbuf[pl.ds(base, 16)]
                plsc.addupdate_scatter(aux, [lab], xbuf[pl.ds(base, 16)])
                plsc.addupdate_scatter(aux, [lab + _S], ybuf[pl.ds(base, 16)])
                plsc.addupdate_scatter(aux, [lab + 2 * _S], fxbuf[pl.ds(base, 16)])
                plsc.addupdate_scatter(aux, [lab + 3 * _S], fybuf[pl.ds(base, 16)])
                plsc.addupdate_scatter(aux, [lab + 4 * _S], ones16)
                return inner
            lax.fori_loop(0, _PCH // 16, ablk, 0, unroll=2)
        return carry

    lax.fori_loop(0, _NCHUNK, chunk_body, 0)

    pltpu.sync_copy(acc, feat_out.at[wid, 0])
    pltpu.sync_copy(aux, aux_out.at[wid, 0])


def _dense_body(feat_ref, aux_ref, cg_ref, cb_ref, c1w_ref, c1b_ref,
                c2w_ref, c2b_ref, brg_ref, brb_ref, srw_ref, srb_ref,
                b5g_ref, b5b_ref, mwx_ref, mwr_ref, mb_ref,
                siam_ref, pos_ref):
    aux = jnp.sum(aux_ref[...], axis=0)
    xx_s, yy_s, fx_s, fy_s, cnt = aux[0], aux[1], aux[2], aux[3], aux[4]
    cts = jnp.maximum(cnt, 1.0)
    xx = xx_s / cts
    yy = yy_s / cts
    fxp = fx_s / cts
    fyp = fy_s / cts

    f = feat_ref[...]
    X = jnp.concatenate([f[0], f[1]], axis=1) / cts[None, :]

    cg = cg_ref[...]
    cb = cb_ref[...]

    def bn1(v, gamma, beta):
        mu = jnp.mean(v)
        var = jnp.mean((v - mu) ** 2)
        return (v - mu) / jnp.sqrt(var + 1e-5) * gamma + beta

    c0 = bn1(xx, cg[0], cb[0])
    c1 = bn1(yy, cg[1], cb[1])
    c2 = bn1(fxp, cg[2], cb[2])
    c3 = bn1(fyp, cg[3], cb[3])

    w1 = c1w_ref[...]
    x1 = (w1[:, 0][:, None] * c0[None, :] + w1[:, 1][:, None] * c1[None, :]
          + w1[:, 2][:, None] * c2[None, :] + w1[:, 3][:, None] * c3[None, :]
          + c1b_ref[...][:, None])
    x2 = jnp.dot(c2w_ref[...], x1, preferred_element_type=jnp.float32)
    x2 = jnp.maximum(x2 + c2b_ref[...][:, None], 0.0)

    mu_c = jnp.mean(X, axis=1)[:, None]
    var_c = jnp.mean((X - mu_c) ** 2, axis=1)[:, None]
    psn = ((X - mu_c) / jnp.sqrt(var_c + 1e-5)
           * brg_ref[...][:, None] + brb_ref[...][:, None])

    r = jnp.dot(srw_ref[...], psn, preferred_element_type=jnp.float32)
    r = r + srb_ref[...][:, None]
    mu_r = jnp.mean(r, axis=1)[:, None]
    var_r = jnp.mean((r - mu_r) ** 2, axis=1)[:, None]
    rn = ((r - mu_r) / jnp.sqrt(var_r + 1e-5)
          * b5g_ref[...][:, None] + b5b_ref[...][:, None])
    rn = jnp.maximum(rn, 0.0)

    siam_t = (jnp.dot(mwx_ref[...], x2, preferred_element_type=jnp.float32)
              + jnp.dot(mwr_ref[...], rn, preferred_element_type=jnp.float32)
              + mb_ref[...][:, None])
    siam_ref[...] = siam_t.T
    pos_ref[...] = jnp.concatenate([xx[:, None], yy[:, None]], axis=1)


_dense_call = pl.pallas_call(
    _dense_body,
    out_shape=[
        jax.ShapeDtypeStruct((_NSEG, _C), jnp.float32),
        jax.ShapeDtypeStruct((_NSEG, 2), jnp.float32),
    ],
)


def kernel(image, fx, fy, autoenc_feats, labels, bn_coords_g, bn_coords_b,
           c1_w, c1_b, c2_w, c2_b, bn_reduc_g, bn_reduc_b, sr_w, sr_b,
           bn512_g, bn512_b, m_w, m_b):
    feats2 = autoenc_feats.reshape(_B * _C, _HW)
    labs = labels.reshape(_B, _HW)
    fxf = fx.reshape(_B, _HW)
    fyf = fy.reshape(_B, _HW)
    ar = jnp.arange(_H, dtype=jnp.float32) / (_H - 1)
    xxv = jnp.broadcast_to(ar[:, None], (_H, _W)).reshape(_HW)
    yyv = jnp.broadcast_to(ar[None, :], (_H, _W)).reshape(_HW)

    feat_sums, aux_raw = _sc_pool(feats2, labs, xxv, yyv, fxf, fyf)
    feat_sums = feat_sums.reshape(_B, _C, _S)
    aux_raw = (aux_raw.reshape(_B, _NG, 5, _S)
               .transpose(1, 2, 0, 3).reshape(_NG, 5, _NSEG))

    siam, pos = _dense_call(
        feat_sums, aux_raw,
        bn_coords_g, bn_coords_b, c1_w, c1_b, c2_w, c2_b,
        bn_reduc_g, bn_reduc_b, sr_w, sr_b, bn512_g, bn512_b,
        m_w[:, :64], m_w[:, 64:], m_b)
    return siam, pos


# speedup vs baseline: 2.8601x; 1.2219x over previous
import functools

import jax
import jax.numpy as jnp
from jax import lax
from jax.experimental import pallas as pl
from jax.experimental.pallas import tpu as pltpu
from jax.experimental.pallas import tpu_sc as plsc

_B = 2
_C = 256
_H = 384
_W = 384
_HW = _H * _W
_S = 1200
_NSEG = _B * _S
_R = 8
_NCHUNK = _H // _R
_NG = 16
_GC = 16
_WB = _W // 16


@functools.partial(
    pl.kernel,
    out_type=[
        jax.ShapeDtypeStruct((_B * _NG, 1, _GC * _S), jnp.float32),
        jax.ShapeDtypeStruct((_B * _NG, 1, 5 * _S), jnp.float32),
    ],
    scratch_types=[
        pltpu.VMEM((_GC, _R, _W), jnp.float32),
        pltpu.VMEM((_R, _W), jnp.int32),
        pltpu.VMEM((_R, _W), jnp.float32),
        pltpu.VMEM((_R, _W), jnp.float32),
        pltpu.VMEM((_GC * _S,), jnp.float32),
        pltpu.VMEM((5 * _S,), jnp.float32),
    ],
    mesh=plsc.VectorSubcoreMesh(core_axis_name="c", subcore_axis_name="s"),
    compiler_params=pltpu.CompilerParams(needs_layout_passes=False),
)
def _sc_pool(feats_hbm, labs_hbm, fx_hbm, fy_hbm,
             feat_out, aux_out,
             fbuf, lbuf, fxbuf, fybuf, acc, aux):
    bb = lax.axis_index("c")
    g = lax.axis_index("s")
    wid = bb * _NG + g
    c0 = g * _GC

    zeros16 = jnp.zeros((16,), jnp.float32)
    ones16 = jnp.ones((16,), jnp.float32)
    iota16 = lax.iota(jnp.int32, 16).astype(jnp.float32) * (1.0 / (_W - 1))

    def _zacc(i, carry):
        acc[pl.ds(i * 16, 16)] = zeros16
        return carry
    lax.fori_loop(0, (_GC * _S) // 16, _zacc, 0, unroll=8)

    def _zaux(i, carry):
        aux[pl.ds(i * 16, 16)] = zeros16
        return carry
    lax.fori_loop(0, (5 * _S) // 16, _zaux, 0, unroll=8)

    def chunk_body(ci, carry):
        r0 = ci * _R
        pltpu.sync_copy(
            feats_hbm.at[bb, pl.ds(c0, _GC), pl.ds(r0, _R), :], fbuf)
        pltpu.sync_copy(labs_hbm.at[bb, pl.ds(r0, _R), :], lbuf)

        def row_loop(ri, rcarry):
            def blk(pj, inner):
                base = pj * 16
                lab = lbuf[ri, pl.ds(base, 16)]
                for c in range(_GC):
                    v = fbuf[c, ri, pl.ds(base, 16)]
                    plsc.addupdate_scatter(acc, [lab + (c * _S)], v)
                return inner
            lax.fori_loop(0, _WB, blk, 0, unroll=4)
            return rcarry
        lax.fori_loop(0, _R, row_loop, 0)

        @pl.when(lax.rem(ci, _NG) == g)
        def _aux_work():
            pltpu.sync_copy(fx_hbm.at[bb, pl.ds(r0, _R), :], fxbuf)
            pltpu.sync_copy(fy_hbm.at[bb, pl.ds(r0, _R), :], fybuf)

            def arow(ri, rcarry):
                xval = (r0 + ri).astype(jnp.float32) * (1.0 / (_H - 1))
                xvec = jnp.full((16,), 1.0, jnp.float32) * xval

                def ablk(pj, inner):
                    base = pj * 16
                    lab = lbuf[ri, pl.ds(base, 16)]
                    yvec = iota16 + jnp.float32(1.0 / (_W - 1)) * base
                    plsc.addupdate_scatter(aux, [lab], xvec)
                    plsc.addupdate_scatter(aux, [lab + _S], yvec)
                    plsc.addupdate_scatter(aux, [lab + 2 * _S],
                                           fxbuf[ri, pl.ds(base, 16)])
                    plsc.addupdate_scatter(aux, [lab + 3 * _S],
                                           fybuf[ri, pl.ds(base, 16)])
                    plsc.addupdate_scatter(aux, [lab + 4 * _S], ones16)
                    return inner
                lax.fori_loop(0, _WB, ablk, 0, unroll=2)
                return rcarry
            lax.fori_loop(0, _R, arow, 0)
        return carry

    lax.fori_loop(0, _NCHUNK, chunk_body, 0)

    pltpu.sync_copy(acc, feat_out.at[wid, 0])
    pltpu.sync_copy(aux, aux_out.at[wid, 0])


def _dense_body(feat_ref, aux_ref, cg_ref, cb_ref, c1w_ref, c1b_ref,
                c2w_ref, c2b_ref, brg_ref, brb_ref, srw_ref, srb_ref,
                b5g_ref, b5b_ref, mwx_ref, mwr_ref, mb_ref,
                siam_ref, pos_ref):
    aux = jnp.sum(aux_ref[...], axis=0)
    xx_s, yy_s, fx_s, fy_s, cnt = aux[0], aux[1], aux[2], aux[3], aux[4]
    cts = jnp.maximum(cnt, 1.0)
    xx = xx_s / cts
    yy = yy_s / cts
    fxp = fx_s / cts
    fyp = fy_s / cts

    f = feat_ref[...]
    X = jnp.concatenate([f[0], f[1]], axis=1) / cts[None, :]

    cg = cg_ref[...]
    cb = cb_ref[...]

    def bn1(v, gamma, beta):
        mu = jnp.mean(v)
        var = jnp.mean((v - mu) ** 2)
        return (v - mu) / jnp.sqrt(var + 1e-5) * gamma + beta

    c0 = bn1(xx, cg[0], cb[0])
    c1 = bn1(yy, cg[1], cb[1])
    c2 = bn1(fxp, cg[2], cb[2])
    c3 = bn1(fyp, cg[3], cb[3])

    w1 = c1w_ref[...]
    x1 = (w1[:, 0][:, None] * c0[None, :] + w1[:, 1][:, None] * c1[None, :]
          + w1[:, 2][:, None] * c2[None, :] + w1[:, 3][:, None] * c3[None, :]
          + c1b_ref[...][:, None])
    x2 = jnp.dot(c2w_ref[...], x1, preferred_element_type=jnp.float32)
    x2 = jnp.maximum(x2 + c2b_ref[...][:, None], 0.0)

    mu_c = jnp.mean(X, axis=1)[:, None]
    var_c = jnp.mean((X - mu_c) ** 2, axis=1)[:, None]
    psn = ((X - mu_c) / jnp.sqrt(var_c + 1e-5)
           * brg_ref[...][:, None] + brb_ref[...][:, None])

    r = jnp.dot(srw_ref[...], psn, preferred_element_type=jnp.float32)
    r = r + srb_ref[...][:, None]
    mu_r = jnp.mean(r, axis=1)[:, None]
    var_r = jnp.mean((r - mu_r) ** 2, axis=1)[:, None]
    rn = ((r - mu_r) / jnp.sqrt(var_r + 1e-5)
          * b5g_ref[...][:, None] + b5b_ref[...][:, None])
    rn = jnp.maximum(rn, 0.0)

    siam_t = (jnp.dot(mwx_ref[...], x2, preferred_element_type=jnp.float32)
              + jnp.dot(mwr_ref[...], rn, preferred_element_type=jnp.float32)
              + mb_ref[...][:, None])
    siam_ref[...] = siam_t.T
    pos_ref[...] = jnp.concatenate([xx[:, None], yy[:, None]], axis=1)


_dense_call = pl.pallas_call(
    _dense_body,
    out_shape=[
        jax.ShapeDtypeStruct((_NSEG, _C), jnp.float32),
        jax.ShapeDtypeStruct((_NSEG, 2), jnp.float32),
    ],
)


def kernel(image, fx, fy, autoenc_feats, labels, bn_coords_g, bn_coords_b,
           c1_w, c1_b, c2_w, c2_b, bn_reduc_g, bn_reduc_b, sr_w, sr_b,
           bn512_g, bn512_b, m_w, m_b):
    labs = labels.reshape(_B, _H, _W)
    fxf = fx.reshape(_B, _H, _W)
    fyf = fy.reshape(_B, _H, _W)

    feat_sums, aux_raw = _sc_pool(autoenc_feats, labs, fxf, fyf)
    feat_sums = feat_sums.reshape(_B, _C, _S)
    aux_raw = (aux_raw.reshape(_B, _NG, 5, _S)
               .transpose(1, 2, 0, 3).reshape(_NG, 5, _NSEG))

    siam, pos = _dense_call(
        feat_sums, aux_raw,
        bn_coords_g, bn_coords_b, c1_w, c1_b, c2_w, c2_b,
        bn_reduc_g, bn_reduc_b, sr_w, sr_b, bn512_g, bn512_b,
        m_w[:, :64], m_w[:, 64:], m_b)
    return siam, pos
